# baseline (speedup 1.0000x reference)
import functools

import jax
import jax.numpy as jnp
from jax import lax
from jax.experimental import pallas as pl
from jax.experimental.pallas import tpu as pltpu
from jax.experimental.pallas import tpu_sc as plsc

N = 50000
NPAD = 51200
PAD = NPAD - N
E = 800000
EP = 851968
EPAD = EP - E
NC, NS = 2, 16
NW = NC * NS
RPS = NPAD // NS
_CH = {1: 1024, 16: 512, 32: 128}
_STAGE = {1: (3200, 1), 16: (800, 4), 32: (200, 16)}

_MESH = plsc.VectorSubcoreMesh(core_axis_name="c", subcore_axis_name="s")


def _wid_cid_sid():
    cid = lax.axis_index("c")
    sid = lax.axis_index("s")
    return sid * NC + cid, cid, sid


def _zero_acc(acc, stage, sid, F, SR, SI):
    nst = (SR * F) // 16

    def zb(i, c):
        if F == 1:
            stage[pl.ds(i * 16, 16)] = jnp.zeros((16,), jnp.float32)
        else:
            stage[0, i // (F // 16), pl.ds((i % (F // 16)) * 16, 16)] = (
                jnp.zeros((16,), jnp.float32))
        return c

    lax.fori_loop(0, nst, zb, 0)
    z = stage if SI == 1 else stage.at[0]
    for i in range(SI):
        pltpu.sync_copy(z, acc.at[pl.ds(sid * RPS + i * SR, SR)])


def _read_acc(acc, stage, out_h, cid, sid, SR, SI, rsem=None):
    if SI == 1:
        pltpu.sync_copy(acc.at[pl.ds(sid * RPS, RPS)], stage)
        pltpu.sync_copy(stage, out_h.at[pl.ds(cid * NPAD + sid * RPS, RPS)])
        return
    off = cid * NPAD + sid * RPS
    for i in range(SI):
        b = i % 2
        if i >= 2:
            pltpu.make_async_copy(
                stage.at[b], out_h.at[pl.ds(off + (i - 2) * SR, SR)],
                rsem[b]).wait()
        pltpu.sync_copy(acc.at[pl.ds(sid * RPS + i * SR, SR)], stage.at[b])
        pltpu.async_copy(stage.at[b], out_h.at[pl.ds(off + i * SR, SR)],
                         rsem[b])
    for i in (SI - 2, SI - 1):
        pltpu.make_async_copy(stage.at[i % 2],
                              out_h.at[pl.ds(off + i * SR, SR)],
                              rsem[i % 2]).wait()


def _run_pipeline(GPW, step, prologue):
    prologue()
    step(0, 0, True, False)
    step(1, 1, False, False)
    step(2, 2, False, False)
    nb = (GPW - 4) // 4

    def gbody(i, carry):
        g0 = 3 + 4 * i
        for s in range(4):
            step(g0 + s, (3 + s) % 4, False, False)
        return carry

    if nb > 0:
        lax.fori_loop(0, nb, gbody, 0)
    for g in range(3 + 4 * nb, GPW - 1):
        step(g, g % 4, False, False)
    step(GPW - 1, (GPW - 1) % 4, False, True)


def _sc_counts(dst2d):
    SR, SI = _STAGE[1]
    CH = _CH[1]
    GPW = EP // (CH * NW)

    @functools.partial(
        pl.kernel,
        out_type=jax.ShapeDtypeStruct((NC * NPAD,), jnp.float32),
        mesh=_MESH,
        compiler_params=pltpu.CompilerParams(use_tc_tiling_on_sc=False),
        scratch_types=[
            pltpu.VMEM_SHARED((NPAD,), jnp.float32),
            pltpu.VMEM((SR,), jnp.float32),
            pltpu.VMEM((4, CH), jnp.int32),
            pltpu.VMEM((CH,), jnp.float32),
        ] + [pltpu.SemaphoreType.DMA] * 8,
    )
    def k(dst_h, out_h, acc, stage, dst_v, ones_v, *sems):
        isem, ssem = sems[:4], sems[4:]
        wid, cid, sid = _wid_cid_sid()
        for i in range(CH // 16):
            ones_v[pl.ds(i * 16, 16)] = jnp.ones((16,), jnp.float32)
        _zero_acc(acc, stage, sid, 1, SR, SI)
        plsc.subcore_barrier()
        base = wid * GPW

        def idx_load(q, g):
            pltpu.async_copy(dst_h.at[base + g], dst_v.at[q], isem[q])

        def idx_wait(q):
            pltpu.make_async_copy(dst_h.at[0], dst_v.at[q], isem[q]).wait()

        def sc_fire(q):
            pltpu.async_copy(ones_v, acc.at[dst_v.at[q]], ssem[q], add=True)

        def sc_wait(q):
            pltpu.make_async_copy(ones_v, acc.at[dst_v.at[q]], ssem[q]).wait()

        def step(g, q, first, last):
            if not last:
                idx_load((q + 1) % 4, g + 1)
            idx_wait(q)
            sc_fire(q)
            if not first:
                sc_wait((q + 3) % 4)

        _run_pipeline(GPW, step, lambda: idx_load(0, 0))
        sc_wait((GPW - 1) % 4)
        plsc.subcore_barrier()
        _read_acc(acc, stage, out_h, cid, sid, SR, SI)

    return k(dst2d)


def _sc_agg(src2d, dst2d, table, F):
    CH = _CH[F]
    GPW = EP // (CH * NW)
    tshape = (NPAD,) if F == 1 else (NPAD, F)
    mshape = (2, CH) if F == 1 else (2, CH, F)
    oshape = (NC * NPAD,) if F == 1 else (NC * NPAD, F)
    SR, SI = _STAGE[F]
    sshape = (SR,) if SI == 1 else (2, SR, F)
    shared = [pltpu.VMEM_SHARED(tshape, jnp.float32)]
    if F == 1:
        shared.append(pltpu.VMEM_SHARED((NPAD,), jnp.float32))

    @functools.partial(
        pl.kernel,
        out_type=jax.ShapeDtypeStruct(oshape, jnp.float32),
        mesh=_MESH,
        compiler_params=pltpu.CompilerParams(use_tc_tiling_on_sc=False),
        scratch_types=shared + [
            pltpu.VMEM(sshape, jnp.float32),
            pltpu.VMEM((4, CH), jnp.int32),
            pltpu.VMEM((4, CH), jnp.int32),
            pltpu.VMEM(mshape, jnp.float32),
        ] + [pltpu.SemaphoreType.DMA] * 14,
    )
    def k(src_h, dst_h, tab_h, out_h, acc, *rest):
        if F == 1:
            tab_s, stage, src_v, dst_v, msg_v = rest[:5]
        else:
            stage, src_v, dst_v, msg_v = rest[:4]
        sems = rest[5:] if F == 1 else rest[4:]
        isem, gsem, ssem, rsem = sems[:4], sems[4:8], sems[8:12], sems[12:]
        wid, cid, sid = _wid_cid_sid()
        if F == 1:
            pltpu.sync_copy(tab_h.at[pl.ds(sid * RPS, RPS)], stage)
            pltpu.sync_copy(stage, tab_s.at[pl.ds(sid * RPS, RPS)])
            tab = tab_s
        else:
            tab = tab_h
        _zero_acc(acc, stage, sid, F, SR, SI)
        plsc.subcore_barrier()
        base = wid * GPW

        def idx_load(q, g):
            pltpu.async_copy(src_h.at[base + g], src_v.at[q], isem[q])
            pltpu.async_copy(dst_h.at[base + g], dst_v.at[q], isem[q])

        def idx_wait(q):
            pltpu.make_async_copy(src_h.at[0], src_v.at[q], isem[q]).wait()
            pltpu.make_async_copy(dst_h.at[0], dst_v.at[q], isem[q]).wait()

        def ga_fire(q, m):
            pltpu.async_copy(tab.at[src_v.at[q]], msg_v.at[m], gsem[q])

        def ga_wait(q, m):
            pltpu.make_async_copy(tab.at[src_v.at[q]], msg_v.at[m],
                                  gsem[q]).wait()

        def sc_fire(q, m):
            pltpu.async_copy(msg_v.at[m], acc.at[dst_v.at[q]], ssem[q],
                             add=True)

        def sc_wait(q, m):
            pltpu.make_async_copy(msg_v.at[m], acc.at[dst_v.at[q]],
                                  ssem[q]).wait()

        def step(g, q, first, last):
            qn, m, mn = (q + 1) % 4, q % 2, (q + 1) % 2
            if not last:
                idx_load(qn, g + 1)
            ga_wait(q, m)
            sc_fire(q, m)
            if not first:
                sc_wait((q + 3) % 4, mn)
            if not last:
                idx_wait(qn)
                ga_fire(qn, mn)

        def prologue():
            idx_load(0, 0)
            idx_wait(0)
            ga_fire(0, 0)

        _run_pipeline(GPW, step, prologue)
        q = (GPW - 1) % 4
        sc_wait(q, q % 2)
        plsc.subcore_barrier()
        _read_acc(acc, stage, out_h, cid, sid, SR, SI, rsem)

    return k(src2d, dst2d, table)


def _d0_body(cntp, x, dinv_o, g1_o):
    c = cntp[...]
    deg = c[0] + c[1] + 1.0
    dinv = lax.rsqrt(deg)
    dinv_o[...] = dinv
    g1_o[...] = dinv * x[...]


def _d1_body(s1p, g1, dinv, W1c, b1c, g2_o):
    s = s1p[...]
    di = dinv[...]
    y = di * (s[0] + s[1] + g1[...])
    h = jnp.maximum(W1c[...] * y + b1c[...], 0.0)
    g2_o[...] = di * h


def _d2_body(s2p, g2, dinv, W2, b2c, g3_o):
    s = s2p[...]
    di = dinv[...]
    a = di * (s[0] + s[1] + g2[...])
    h = lax.dot_general(W2[...], a, (((0,), (0,)), ((), ())),
                        preferred_element_type=jnp.float32)
    h = jnp.maximum(h + b2c[...], 0.0)
    g3_o[...] = di * h


def _d3a_body(s3p, g3, dinv, W3, b3c, h3_o):
    s = s3p[...]
    di = dinv[...]
    a = di * (s[0] + s[1] + g3[...])
    h = lax.dot_general(W3[...], a, (((0,), (0,)), ((), ())),
                        preferred_element_type=jnp.float32)
    h3_o[...] = jnp.maximum(h + b3c[...], 0.0)


def _d3b_body(h3, batch, Wfc, bfc, out):
    seg = lax.broadcasted_iota(jnp.int32, (64, 1), 0)
    B = (batch[...] == seg).astype(jnp.float32)
    sums = lax.dot_general(h3[...], B, (((1,), (1,)), ((), ())),
                           preferred_element_type=jnp.float32)
    ones = jnp.ones((1, NPAD), jnp.float32)
    cnts = lax.dot_general(ones, B, (((1,), (1,)), ((), ())),
                           preferred_element_type=jnp.float32)
    pooled = sums / jnp.maximum(cnts, 1.0)
    logits = lax.dot_general(pooled, Wfc[...], (((0,), (0,)), ((), ())),
                             preferred_element_type=jnp.float32) + bfc[...]
    m = jnp.max(logits, axis=1, keepdims=True)
    z = logits - m
    lse = jnp.log(jnp.sum(jnp.exp(z), axis=1, keepdims=True))
    out[...] = z - lse


def _tc(body, out_shape, *args):
    return pl.pallas_call(body, out_shape=out_shape)(*args)


def kernel(x, edge_index, batch, W1, b1, W2, b2, W3, b3, Wfc, bfc):
    ar = jnp.arange(EPAD, dtype=jnp.int32)
    srcp = jnp.concatenate([edge_index[0], ar % NPAD])
    dstp = jnp.concatenate([edge_index[1], N + (ar % PAD)])
    sv = {c: srcp.reshape(EP // c, c) for c in set(_CH.values())}
    dv = {c: dstp.reshape(EP // c, c) for c in set(_CH.values())}
    f32 = jnp.float32
    sds = jax.ShapeDtypeStruct

    cntp = _sc_counts(dv[_CH[1]])
    xp = jnp.pad(x[:, 0], (0, PAD)).reshape(1, NPAD)
    dinv, g1 = _tc(_d0_body,
                   (sds((1, NPAD), f32), sds((1, NPAD), f32)),
                   cntp.reshape(NC, 1, NPAD), xp)
    s1p = _sc_agg(sv[_CH[1]], dv[_CH[1]], g1.reshape(NPAD), 1)
    g2f = _tc(_d1_body, sds((16, NPAD), f32),
              s1p.reshape(NC, 1, NPAD), g1, dinv,
              W1.reshape(16, 1), b1.reshape(16, 1))
    s2p = _sc_agg(sv[_CH[16]], dv[_CH[16]], g2f.T, 16)
    g3f = _tc(_d2_body, sds((32, NPAD), f32),
              s2p.reshape(NC, NPAD, 16).transpose(0, 2, 1), g2f, dinv,
              W2, b2.reshape(32, 1))
    s3p = _sc_agg(sv[_CH[32]], dv[_CH[32]], g3f.T, 32)
    h3f = _tc(_d3a_body, sds((64, NPAD), f32),
              s3p.reshape(NC, NPAD, 32).transpose(0, 2, 1), g3f, dinv,
              W3, b3.reshape(64, 1))
    bp = jnp.pad(batch, (0, PAD), constant_values=64).reshape(1, NPAD)
    out = _tc(_d3b_body, sds((64, 4), f32),
              h3f, bp, Wfc, bfc.reshape(1, 4))
    return out


# speedup vs baseline: 1.1040x; 1.1040x over previous
import functools

import jax
import jax.numpy as jnp
from jax import lax
from jax.experimental import pallas as pl
from jax.experimental.pallas import tpu as pltpu
from jax.experimental.pallas import tpu_sc as plsc

N = 50000
NPAD = 51200
PAD = NPAD - N
E = 800000
EP = 851968
EPAD = EP - E
NC, NS = 2, 16
NW = NC * NS
RPS = NPAD // NS
_CH = {1: 1024, 16: 512, 32: 256}
_STAGE = {1: (3200, 1), 16: (800, 4), 32: (80, 40)}

_MESH = plsc.VectorSubcoreMesh(core_axis_name="c", subcore_axis_name="s")


def _wid_cid_sid():
    cid = lax.axis_index("c")
    sid = lax.axis_index("s")
    return sid * NC + cid, cid, sid


def _zero_acc(acc, stage, sid, F, SR, SI):
    nst = (SR * F) // 16

    def zb(i, c):
        if F == 1:
            stage[pl.ds(i * 16, 16)] = jnp.zeros((16,), jnp.float32)
        else:
            stage[0, i // (F // 16), pl.ds((i % (F // 16)) * 16, 16)] = (
                jnp.zeros((16,), jnp.float32))
        return c

    lax.fori_loop(0, nst, zb, 0)
    z = stage if SI == 1 else stage.at[0]
    for i in range(SI):
        pltpu.sync_copy(z, acc.at[pl.ds(sid * RPS + i * SR, SR)])


def _read_acc(acc, stage, out_h, cid, sid, SR, SI, rsem=None):
    if SI == 1:
        pltpu.sync_copy(acc.at[pl.ds(sid * RPS, RPS)], stage)
        pltpu.sync_copy(stage, out_h.at[pl.ds(cid * NPAD + sid * RPS, RPS)])
        return
    off = cid * NPAD + sid * RPS
    for i in range(SI):
        b = i % 2
        if i >= 2:
            pltpu.make_async_copy(
                stage.at[b], out_h.at[pl.ds(off + (i - 2) * SR, SR)],
                rsem[b]).wait()
        pltpu.sync_copy(acc.at[pl.ds(sid * RPS + i * SR, SR)], stage.at[b])
        pltpu.async_copy(stage.at[b], out_h.at[pl.ds(off + i * SR, SR)],
                         rsem[b])
    for i in (SI - 2, SI - 1):
        pltpu.make_async_copy(stage.at[i % 2],
                              out_h.at[pl.ds(off + i * SR, SR)],
                              rsem[i % 2]).wait()


def _run_pipeline(GPW, step, prologue):
    prologue()
    step(0, 0, True, False)
    step(1, 1, False, False)
    step(2, 2, False, False)
    nb = (GPW - 4) // 4

    def gbody(i, carry):
        g0 = 3 + 4 * i
        for s in range(4):
            step(g0 + s, (3 + s) % 4, False, False)
        return carry

    if nb > 0:
        lax.fori_loop(0, nb, gbody, 0)
    for g in range(3 + 4 * nb, GPW - 1):
        step(g, g % 4, False, False)
    step(GPW - 1, (GPW - 1) % 4, False, True)


def _sc_counts(dst2d):
    SR, SI = _STAGE[1]
    CH = _CH[1]
    GPW = EP // (CH * NW)

    @functools.partial(
        pl.kernel,
        out_type=jax.ShapeDtypeStruct((NC * NPAD,), jnp.float32),
        mesh=_MESH,
        compiler_params=pltpu.CompilerParams(use_tc_tiling_on_sc=False),
        scratch_types=[
            pltpu.VMEM_SHARED((NPAD,), jnp.float32),
            pltpu.VMEM((SR,), jnp.float32),
            pltpu.VMEM((4, CH), jnp.int32),
            pltpu.VMEM((CH,), jnp.float32),
        ] + [pltpu.SemaphoreType.DMA] * 8,
    )
    def k(dst_h, out_h, acc, stage, dst_v, ones_v, *sems):
        isem, ssem = sems[:4], sems[4:]
        wid, cid, sid = _wid_cid_sid()
        for i in range(CH // 16):
            ones_v[pl.ds(i * 16, 16)] = jnp.ones((16,), jnp.float32)
        _zero_acc(acc, stage, sid, 1, SR, SI)
        plsc.subcore_barrier()
        base = wid * GPW

        def idx_load(q, g):
            pltpu.async_copy(dst_h.at[base + g], dst_v.at[q], isem[q])

        def idx_wait(q):
            pltpu.make_async_copy(dst_h.at[0], dst_v.at[q], isem[q]).wait()

        def sc_fire(q):
            pltpu.async_copy(ones_v, acc.at[dst_v.at[q]], ssem[q], add=True)

        def sc_wait(q):
            pltpu.make_async_copy(ones_v, acc.at[dst_v.at[q]], ssem[q]).wait()

        def step(g, q, first, last):
            if not last:
                idx_load((q + 1) % 4, g + 1)
            idx_wait(q)
            sc_fire(q)
            if not first:
                sc_wait((q + 3) % 4)

        _run_pipeline(GPW, step, lambda: idx_load(0, 0))
        sc_wait((GPW - 1) % 4)
        plsc.subcore_barrier()
        _read_acc(acc, stage, out_h, cid, sid, SR, SI)

    return k(dst2d)


def _sc_agg(src2d, dst2d, table, F):
    CH = _CH[F]
    GPW = EP // (CH * NW)
    tshape = (NPAD,) if F == 1 else (NPAD, F)
    mshape = (2, CH) if F == 1 else (2, CH, F)
    oshape = (NC * NPAD,) if F == 1 else (NC * NPAD, F)
    SR, SI = _STAGE[F]
    sshape = (SR,) if SI == 1 else (2, SR, F)
    shared = [pltpu.VMEM_SHARED(tshape, jnp.float32)]
    if F == 1:
        shared.append(pltpu.VMEM_SHARED((NPAD,), jnp.float32))

    @functools.partial(
        pl.kernel,
        out_type=jax.ShapeDtypeStruct(oshape, jnp.float32),
        mesh=_MESH,
        compiler_params=pltpu.CompilerParams(use_tc_tiling_on_sc=False),
        scratch_types=shared + [
            pltpu.VMEM(sshape, jnp.float32),
            pltpu.VMEM((4, CH), jnp.int32),
            pltpu.VMEM((4, CH), jnp.int32),
            pltpu.VMEM(mshape, jnp.float32),
        ] + [pltpu.SemaphoreType.DMA] * 14,
    )
    def k(src_h, dst_h, tab_h, out_h, acc, *rest):
        if F == 1:
            tab_s, stage, src_v, dst_v, msg_v = rest[:5]
        else:
            stage, src_v, dst_v, msg_v = rest[:4]
        sems = rest[5:] if F == 1 else rest[4:]
        isem, gsem, ssem, rsem = sems[:4], sems[4:8], sems[8:12], sems[12:]
        wid, cid, sid = _wid_cid_sid()
        if F == 1:
            pltpu.sync_copy(tab_h.at[pl.ds(sid * RPS, RPS)], stage)
            pltpu.sync_copy(stage, tab_s.at[pl.ds(sid * RPS, RPS)])
            tab = tab_s
        else:
            tab = tab_h
        _zero_acc(acc, stage, sid, F, SR, SI)
        plsc.subcore_barrier()
        base = wid * GPW

        def idx_load(q, g):
            pltpu.async_copy(src_h.at[base + g], src_v.at[q], isem[q])
            pltpu.async_copy(dst_h.at[base + g], dst_v.at[q], isem[q])

        def idx_wait(q):
            pltpu.make_async_copy(src_h.at[0], src_v.at[q], isem[q]).wait()
            pltpu.make_async_copy(dst_h.at[0], dst_v.at[q], isem[q]).wait()

        def ga_fire(q, m):
            pltpu.async_copy(tab.at[src_v.at[q]], msg_v.at[m], gsem[q])

        def ga_wait(q, m):
            pltpu.make_async_copy(tab.at[src_v.at[q]], msg_v.at[m],
                                  gsem[q]).wait()

        def sc_fire(q, m):
            pltpu.async_copy(msg_v.at[m], acc.at[dst_v.at[q]], ssem[q],
                             add=True)

        def sc_wait(q, m):
            pltpu.make_async_copy(msg_v.at[m], acc.at[dst_v.at[q]],
                                  ssem[q]).wait()

        def step(g, q, first, last):
            qn, m, mn = (q + 1) % 4, q % 2, (q + 1) % 2
            if not last:
                idx_load(qn, g + 1)
            ga_wait(q, m)
            sc_fire(q, m)
            if not first:
                sc_wait((q + 3) % 4, mn)
            if not last:
                idx_wait(qn)
                ga_fire(qn, mn)

        def prologue():
            idx_load(0, 0)
            idx_wait(0)
            ga_fire(0, 0)

        _run_pipeline(GPW, step, prologue)
        q = (GPW - 1) % 4
        sc_wait(q, q % 2)
        plsc.subcore_barrier()
        _read_acc(acc, stage, out_h, cid, sid, SR, SI, rsem)

    return k(src2d, dst2d, table)


def _d0_body(cntp, x, dinv_o, g1_o):
    c = cntp[...]
    deg = c[0] + c[1] + 1.0
    dinv = lax.rsqrt(deg)
    dinv_o[...] = dinv
    g1_o[...] = dinv * x[...]


def _d1_body(s1p, g1, dinv, W1c, b1c, g2_o):
    s = s1p[...]
    di = dinv[...]
    y = di * (s[0] + s[1] + g1[...])
    h = jnp.maximum(W1c[...] * y + b1c[...], 0.0)
    g2_o[...] = di * h


def _d2_body(s2p, g2, dinv, W2, b2c, g3_o):
    s = s2p[...]
    di = dinv[...]
    a = di * (s[0] + s[1] + g2[...])
    h = lax.dot_general(W2[...], a, (((0,), (0,)), ((), ())),
                        preferred_element_type=jnp.float32)
    h = jnp.maximum(h + b2c[...], 0.0)
    g3_o[...] = di * h


def _d3a_body(s3p, g3, dinv, W3, b3c, h3_o):
    s = s3p[...]
    di = dinv[...]
    a = di * (s[0] + s[1] + g3[...])
    h = lax.dot_general(W3[...], a, (((0,), (0,)), ((), ())),
                        preferred_element_type=jnp.float32)
    h3_o[...] = jnp.maximum(h + b3c[...], 0.0)


def _d3b_body(h3, batch, Wfc, bfc, out):
    seg = lax.broadcasted_iota(jnp.int32, (64, 1), 0)
    B = (batch[...] == seg).astype(jnp.float32)
    sums = lax.dot_general(h3[...], B, (((1,), (1,)), ((), ())),
                           preferred_element_type=jnp.float32)
    ones = jnp.ones((1, NPAD), jnp.float32)
    cnts = lax.dot_general(ones, B, (((1,), (1,)), ((), ())),
                           preferred_element_type=jnp.float32)
    pooled = sums / jnp.maximum(cnts, 1.0)
    logits = lax.dot_general(pooled, Wfc[...], (((0,), (0,)), ((), ())),
                             preferred_element_type=jnp.float32) + bfc[...]
    m = jnp.max(logits, axis=1, keepdims=True)
    z = logits - m
    lse = jnp.log(jnp.sum(jnp.exp(z), axis=1, keepdims=True))
    out[...] = z - lse


def _tc(body, out_shape, *args):
    return pl.pallas_call(body, out_shape=out_shape)(*args)


def kernel(x, edge_index, batch, W1, b1, W2, b2, W3, b3, Wfc, bfc):
    ar = jnp.arange(EPAD, dtype=jnp.int32)
    srcp = jnp.concatenate([edge_index[0], ar % NPAD])
    dstp = jnp.concatenate([edge_index[1], N + (ar % PAD)])
    sv = {c: srcp.reshape(EP // c, c) for c in set(_CH.values())}
    dv = {c: dstp.reshape(EP // c, c) for c in set(_CH.values())}
    f32 = jnp.float32
    sds = jax.ShapeDtypeStruct

    cntp = _sc_counts(dv[_CH[1]])
    xp = jnp.pad(x[:, 0], (0, PAD)).reshape(1, NPAD)
    dinv, g1 = _tc(_d0_body,
                   (sds((1, NPAD), f32), sds((1, NPAD), f32)),
                   cntp.reshape(NC, 1, NPAD), xp)
    s1p = _sc_agg(sv[_CH[1]], dv[_CH[1]], g1.reshape(NPAD), 1)
    g2f = _tc(_d1_body, sds((16, NPAD), f32),
              s1p.reshape(NC, 1, NPAD), g1, dinv,
              W1.reshape(16, 1), b1.reshape(16, 1))
    s2p = _sc_agg(sv[_CH[16]], dv[_CH[16]], g2f.T, 16)
    g3f = _tc(_d2_body, sds((32, NPAD), f32),
              s2p.reshape(NC, NPAD, 16).transpose(0, 2, 1), g2f, dinv,
              W2, b2.reshape(32, 1))
    s3p = _sc_agg(sv[_CH[32]], dv[_CH[32]], g3f.T, 32)
    h3f = _tc(_d3a_body, sds((64, NPAD), f32),
              s3p.reshape(NC, NPAD, 32).transpose(0, 2, 1), g3f, dinv,
              W3, b3.reshape(64, 1))
    bp = jnp.pad(batch, (0, PAD), constant_values=64).reshape(1, NPAD)
    out = _tc(_d3b_body, sds((64, 4), f32),
              h3f, bp, Wfc, bfc.reshape(1, 4))
    return out


# speedup vs baseline: 1.1802x; 1.0690x over previous
import functools

import jax
import jax.numpy as jnp
from jax import lax
from jax.experimental import pallas as pl
from jax.experimental.pallas import tpu as pltpu
from jax.experimental.pallas import tpu_sc as plsc

N = 50000
NPAD = 51200
PAD = NPAD - N
E = 800000
EP = 851968
EPAD = EP - E
NC, NS = 2, 16
NW = NC * NS
RPS = NPAD // NS
_CH = {1: 1024, 16: 1024, 32: 256}
_STAGE = {1: (3200, 1), 16: (800, 4), 32: (80, 40)}

_MESH = plsc.VectorSubcoreMesh(core_axis_name="c", subcore_axis_name="s")


def _wid_cid_sid():
    cid = lax.axis_index("c")
    sid = lax.axis_index("s")
    return sid * NC + cid, cid, sid


def _zero_acc(acc, stage, sid, F, SR, SI):
    nst = (SR * F) // 16

    def zb(i, c):
        if F == 1:
            stage[pl.ds(i * 16, 16)] = jnp.zeros((16,), jnp.float32)
        else:
            stage[0, i // (F // 16), pl.ds((i % (F // 16)) * 16, 16)] = (
                jnp.zeros((16,), jnp.float32))
        return c

    lax.fori_loop(0, nst, zb, 0)
    z = stage if SI == 1 else stage.at[0]
    for i in range(SI):
        pltpu.sync_copy(z, acc.at[pl.ds(sid * RPS + i * SR, SR)])


def _read_acc(acc, stage, out_h, cid, sid, SR, SI, rsem=None):
    if SI == 1:
        pltpu.sync_copy(acc.at[pl.ds(sid * RPS, RPS)], stage)
        pltpu.sync_copy(stage, out_h.at[pl.ds(cid * NPAD + sid * RPS, RPS)])
        return
    off = cid * NPAD + sid * RPS
    for i in range(SI):
        b = i % 2
        if i >= 2:
            pltpu.make_async_copy(
                stage.at[b], out_h.at[pl.ds(off + (i - 2) * SR, SR)],
                rsem[b]).wait()
        pltpu.sync_copy(acc.at[pl.ds(sid * RPS + i * SR, SR)], stage.at[b])
        pltpu.async_copy(stage.at[b], out_h.at[pl.ds(off + i * SR, SR)],
                         rsem[b])
    for i in (SI - 2, SI - 1):
        pltpu.make_async_copy(stage.at[i % 2],
                              out_h.at[pl.ds(off + i * SR, SR)],
                              rsem[i % 2]).wait()


def _run_pipeline(GPW, step, prologue):
    prologue()
    step(0, 0, True, False)
    step(1, 1, False, False)
    step(2, 2, False, False)
    nb = (GPW - 4) // 4

    def gbody(i, carry):
        g0 = 3 + 4 * i
        for s in range(4):
            step(g0 + s, (3 + s) % 4, False, False)
        return carry

    if nb > 0:
        lax.fori_loop(0, nb, gbody, 0)
    for g in range(3 + 4 * nb, GPW - 1):
        step(g, g % 4, False, False)
    step(GPW - 1, (GPW - 1) % 4, False, True)


def _sc_counts(dst2d):
    SR, SI = _STAGE[1]
    CH = _CH[1]
    GPW = EP // (CH * NW)

    @functools.partial(
        pl.kernel,
        out_type=jax.ShapeDtypeStruct((NC * NPAD,), jnp.float32),
        mesh=_MESH,
        compiler_params=pltpu.CompilerParams(use_tc_tiling_on_sc=False),
        scratch_types=[
            pltpu.VMEM_SHARED((NPAD,), jnp.float32),
            pltpu.VMEM((SR,), jnp.float32),
            pltpu.VMEM((4, CH), jnp.int32),
            pltpu.VMEM((CH,), jnp.float32),
        ] + [pltpu.SemaphoreType.DMA] * 8,
    )
    def k(dst_h, out_h, acc, stage, dst_v, ones_v, *sems):
        isem, ssem = sems[:4], sems[4:]
        wid, cid, sid = _wid_cid_sid()
        for i in range(CH // 16):
            ones_v[pl.ds(i * 16, 16)] = jnp.ones((16,), jnp.float32)
        _zero_acc(acc, stage, sid, 1, SR, SI)
        plsc.subcore_barrier()
        base = wid * GPW

        def idx_load(q, g):
            pltpu.async_copy(dst_h.at[base + g], dst_v.at[q], isem[q])

        def idx_wait(q):
            pltpu.make_async_copy(dst_h.at[0], dst_v.at[q], isem[q]).wait()

        def sc_fire(q):
            pltpu.async_copy(ones_v, acc.at[dst_v.at[q]], ssem[q], add=True)

        def sc_wait(q):
            pltpu.make_async_copy(ones_v, acc.at[dst_v.at[q]], ssem[q]).wait()

        def step(g, q, first, last):
            if not last:
                idx_load((q + 1) % 4, g + 1)
            idx_wait(q)
            sc_fire(q)
            if not first:
                sc_wait((q + 3) % 4)

        _run_pipeline(GPW, step, lambda: idx_load(0, 0))
        sc_wait((GPW - 1) % 4)
        plsc.subcore_barrier()
        _read_acc(acc, stage, out_h, cid, sid, SR, SI)

    return k(dst2d)


def _sc_agg(src2d, dst2d, table, F):
    CH = _CH[F]
    GPW = EP // (CH * NW)
    tshape = (NPAD,) if F == 1 else (NPAD, F)
    mshape = (2, CH) if F == 1 else (2, CH, F)
    oshape = (NC * NPAD,) if F == 1 else (NC * NPAD, F)
    SR, SI = _STAGE[F]
    sshape = (SR,) if SI == 1 else (2, SR, F)
    shared = [pltpu.VMEM_SHARED(tshape, jnp.float32)]
    if F == 1:
        shared.append(pltpu.VMEM_SHARED((NPAD,), jnp.float32))

    @functools.partial(
        pl.kernel,
        out_type=jax.ShapeDtypeStruct(oshape, jnp.float32),
        mesh=_MESH,
        compiler_params=pltpu.CompilerParams(use_tc_tiling_on_sc=False),
        scratch_types=shared + [
            pltpu.VMEM(sshape, jnp.float32),
            pltpu.VMEM((4, CH), jnp.int32),
            pltpu.VMEM((4, CH), jnp.int32),
            pltpu.VMEM(mshape, jnp.float32),
        ] + [pltpu.SemaphoreType.DMA] * 14,
    )
    def k(src_h, dst_h, tab_h, out_h, acc, *rest):
        if F == 1:
            tab_s, stage, src_v, dst_v, msg_v = rest[:5]
        else:
            stage, src_v, dst_v, msg_v = rest[:4]
        sems = rest[5:] if F == 1 else rest[4:]
        isem, gsem, ssem, rsem = sems[:4], sems[4:8], sems[8:12], sems[12:]
        wid, cid, sid = _wid_cid_sid()
        if F == 1:
            pltpu.sync_copy(tab_h.at[pl.ds(sid * RPS, RPS)], stage)
            pltpu.sync_copy(stage, tab_s.at[pl.ds(sid * RPS, RPS)])
            tab = tab_s
        else:
            tab = tab_h
        _zero_acc(acc, stage, sid, F, SR, SI)
        plsc.subcore_barrier()
        base = wid * GPW

        def idx_load(q, g):
            pltpu.async_copy(src_h.at[base + g], src_v.at[q], isem[q])
            pltpu.async_copy(dst_h.at[base + g], dst_v.at[q], isem[q])

        def idx_wait(q):
            pltpu.make_async_copy(src_h.at[0], src_v.at[q], isem[q]).wait()
            pltpu.make_async_copy(dst_h.at[0], dst_v.at[q], isem[q]).wait()

        def ga_fire(q, m):
            pltpu.async_copy(tab.at[src_v.at[q]], msg_v.at[m], gsem[q])

        def ga_wait(q, m):
            pltpu.make_async_copy(tab.at[src_v.at[q]], msg_v.at[m],
                                  gsem[q]).wait()

        def sc_fire(q, m):
            pltpu.async_copy(msg_v.at[m], acc.at[dst_v.at[q]], ssem[q],
                             add=True)

        def sc_wait(q, m):
            pltpu.make_async_copy(msg_v.at[m], acc.at[dst_v.at[q]],
                                  ssem[q]).wait()

        def step(g, q, first, last):
            qn, m, mn = (q + 1) % 4, q % 2, (q + 1) % 2
            if not last:
                idx_load(qn, g + 1)
            ga_wait(q, m)
            sc_fire(q, m)
            if not first:
                sc_wait((q + 3) % 4, mn)
            if not last:
                idx_wait(qn)
                ga_fire(qn, mn)

        def prologue():
            idx_load(0, 0)
            idx_wait(0)
            ga_fire(0, 0)

        _run_pipeline(GPW, step, prologue)
        q = (GPW - 1) % 4
        sc_wait(q, q % 2)
        plsc.subcore_barrier()
        _read_acc(acc, stage, out_h, cid, sid, SR, SI, rsem)

    return k(src2d, dst2d, table)


def _d0_body(cntp, x, dinv_o, g1_o):
    c = cntp[...]
    deg = c[0] + c[1] + 1.0
    dinv = lax.rsqrt(deg)
    dinv_o[...] = dinv
    g1_o[...] = dinv * x[...]


def _d1_body(s1p, g1, dinv, W1c, b1c, g2_o):
    s = s1p[...]
    di = dinv[...]
    y = di * (s[0] + s[1] + g1[...])
    h = jnp.maximum(W1c[...] * y + b1c[...], 0.0)
    g2_o[...] = di * h


def _d2_body(s2p, g2, dinv, W2, b2c, g3_o):
    s = s2p[...]
    di = dinv[...]
    a = di * (s[0] + s[1] + g2[...])
    h = lax.dot_general(W2[...], a, (((0,), (0,)), ((), ())),
                        preferred_element_type=jnp.float32)
    h = jnp.maximum(h + b2c[...], 0.0)
    g3_o[...] = di * h


def _d3_body(s3p, g3, dinv, W3, b3c, batch, Wfc, bfc, out):
    s = s3p[...]
    di = dinv[...]
    a = di * (s[0] + s[1] + g3[...])
    h = lax.dot_general(W3[...], a, (((0,), (0,)), ((), ())),
                        preferred_element_type=jnp.float32)
    h3 = jnp.maximum(h + b3c[...], 0.0)
    seg = lax.broadcasted_iota(jnp.int32, (64, 1), 0)
    B = (batch[...] == seg).astype(jnp.float32)
    sums = lax.dot_general(h3, B, (((1,), (1,)), ((), ())),
                           preferred_element_type=jnp.float32)
    ones = jnp.ones((1, NPAD), jnp.float32)
    cnts = lax.dot_general(ones, B, (((1,), (1,)), ((), ())),
                           preferred_element_type=jnp.float32)
    pooled = sums / jnp.maximum(cnts, 1.0)
    logits = lax.dot_general(pooled, Wfc[...], (((0,), (0,)), ((), ())),
                             preferred_element_type=jnp.float32) + bfc[...]
    m = jnp.max(logits, axis=1, keepdims=True)
    z = logits - m
    lse = jnp.log(jnp.sum(jnp.exp(z), axis=1, keepdims=True))
    out[...] = z - lse


def _tc(body, out_shape, *args):
    return pl.pallas_call(body, out_shape=out_shape)(*args)


def kernel(x, edge_index, batch, W1, b1, W2, b2, W3, b3, Wfc, bfc):
    ar = jnp.arange(EPAD, dtype=jnp.int32)
    srcp = jnp.concatenate([edge_index[0], ar % NPAD])
    dstp = jnp.concatenate([edge_index[1], N + (ar % PAD)])
    sv = {c: srcp.reshape(EP // c, c) for c in set(_CH.values())}
    dv = {c: dstp.reshape(EP // c, c) for c in set(_CH.values())}
    f32 = jnp.float32
    sds = jax.ShapeDtypeStruct

    cntp = _sc_counts(dv[_CH[1]])
    xp = jnp.pad(x[:, 0], (0, PAD)).reshape(1, NPAD)
    dinv, g1 = _tc(_d0_body,
                   (sds((1, NPAD), f32), sds((1, NPAD), f32)),
                   cntp.reshape(NC, 1, NPAD), xp)
    s1p = _sc_agg(sv[_CH[1]], dv[_CH[1]], g1.reshape(NPAD), 1)
    g2f = _tc(_d1_body, sds((16, NPAD), f32),
              s1p.reshape(NC, 1, NPAD), g1, dinv,
              W1.reshape(16, 1), b1.reshape(16, 1))
    s2p = _sc_agg(sv[_CH[16]], dv[_CH[16]], g2f.T, 16)
    g3f = _tc(_d2_body, sds((32, NPAD), f32),
              s2p.reshape(NC, NPAD, 16).transpose(0, 2, 1), g2f, dinv,
              W2, b2.reshape(32, 1))
    s3p = _sc_agg(sv[_CH[32]], dv[_CH[32]], g3f.T, 32)
    bp = jnp.pad(batch, (0, PAD), constant_values=64).reshape(1, NPAD)
    out = _tc(_d3_body, sds((64, 4), f32),
              s3p.reshape(NC, NPAD, 32).transpose(0, 2, 1), g3f, dinv,
              W3, b3.reshape(64, 1), bp, Wfc, bfc.reshape(1, 4))
    return out


# speedup vs baseline: 1.2080x; 1.0236x over previous
import functools

import jax
import jax.numpy as jnp
from jax import lax
from jax.experimental import pallas as pl
from jax.experimental.pallas import tpu as pltpu
from jax.experimental.pallas import tpu_sc as plsc

N = 50000
NPAD = 51200
PAD = NPAD - N
E = 800000
EP = 851968
EPAD = EP - E
NC, NS = 2, 16
NW = NC * NS
RPS = NPAD // NS
_CH = {1: 1024, 16: 1024, 32: 1024}
_STAGE = {1: (3200, 1), 16: (800, 4), 32: (80, 40)}

_MESH = plsc.VectorSubcoreMesh(core_axis_name="c", subcore_axis_name="s")


def _wid_cid_sid():
    cid = lax.axis_index("c")
    sid = lax.axis_index("s")
    return sid * NC + cid, cid, sid


def _zero_acc(acc, stage, sid, F, SR, SI):
    nst = (SR * F) // 16

    def zb(i, c):
        if F == 1:
            stage[pl.ds(i * 16, 16)] = jnp.zeros((16,), jnp.float32)
        else:
            stage[0, i // (F // 16), pl.ds((i % (F // 16)) * 16, 16)] = (
                jnp.zeros((16,), jnp.float32))
        return c

    lax.fori_loop(0, nst, zb, 0)
    z = stage if SI == 1 else stage.at[0]
    for i in range(SI):
        pltpu.sync_copy(z, acc.at[pl.ds(sid * RPS + i * SR, SR)])


def _read_acc(acc, stage, out_h, cid, sid, SR, SI, rsem=None):
    if SI == 1:
        pltpu.sync_copy(acc.at[pl.ds(sid * RPS, RPS)], stage)
        pltpu.sync_copy(stage, out_h.at[pl.ds(cid * NPAD + sid * RPS, RPS)])
        return
    off = cid * NPAD + sid * RPS
    for i in range(SI):
        b = i % 2
        if i >= 2:
            pltpu.make_async_copy(
                stage.at[b], out_h.at[pl.ds(off + (i - 2) * SR, SR)],
                rsem[b]).wait()
        pltpu.sync_copy(acc.at[pl.ds(sid * RPS + i * SR, SR)], stage.at[b])
        pltpu.async_copy(stage.at[b], out_h.at[pl.ds(off + i * SR, SR)],
                         rsem[b])
    for i in (SI - 2, SI - 1):
        pltpu.make_async_copy(stage.at[i % 2],
                              out_h.at[pl.ds(off + i * SR, SR)],
                              rsem[i % 2]).wait()


def _run_pipeline(GPW, step, prologue):
    prologue()
    step(0, 0, True, False)
    step(1, 1, False, False)
    step(2, 2, False, False)
    nb = (GPW - 4) // 4

    def gbody(i, carry):
        g0 = 3 + 4 * i
        for s in range(4):
            step(g0 + s, (3 + s) % 4, False, False)
        return carry

    if nb > 0:
        lax.fori_loop(0, nb, gbody, 0)
    for g in range(3 + 4 * nb, GPW - 1):
        step(g, g % 4, False, False)
    step(GPW - 1, (GPW - 1) % 4, False, True)


def _sc_counts(dst2d):
    SR, SI = _STAGE[1]
    CH = _CH[1]
    GPW = EP // (CH * NW)

    @functools.partial(
        pl.kernel,
        out_type=jax.ShapeDtypeStruct((NC * NPAD,), jnp.float32),
        mesh=_MESH,
        compiler_params=pltpu.CompilerParams(use_tc_tiling_on_sc=False),
        scratch_types=[
            pltpu.VMEM_SHARED((NPAD,), jnp.float32),
            pltpu.VMEM((SR,), jnp.float32),
            pltpu.VMEM((4, CH), jnp.int32),
            pltpu.VMEM((CH,), jnp.float32),
        ] + [pltpu.SemaphoreType.DMA] * 8,
    )
    def k(dst_h, out_h, acc, stage, dst_v, ones_v, *sems):
        isem, ssem = sems[:4], sems[4:]
        wid, cid, sid = _wid_cid_sid()
        for i in range(CH // 16):
            ones_v[pl.ds(i * 16, 16)] = jnp.ones((16,), jnp.float32)
        _zero_acc(acc, stage, sid, 1, SR, SI)
        plsc.subcore_barrier()
        base = wid * GPW

        def idx_load(q, g):
            pltpu.async_copy(dst_h.at[base + g], dst_v.at[q], isem[q])

        def idx_wait(q):
            pltpu.make_async_copy(dst_h.at[0], dst_v.at[q], isem[q]).wait()

        def sc_fire(q):
            pltpu.async_copy(ones_v, acc.at[dst_v.at[q]], ssem[q], add=True)

        def sc_wait(q):
            pltpu.make_async_copy(ones_v, acc.at[dst_v.at[q]], ssem[q]).wait()

        def step(g, q, first, last):
            if not last:
                idx_load((q + 1) % 4, g + 1)
            idx_wait(q)
            sc_fire(q)
            if not first:
                sc_wait((q + 3) % 4)

        _run_pipeline(GPW, step, lambda: idx_load(0, 0))
        sc_wait((GPW - 1) % 4)
        plsc.subcore_barrier()
        _read_acc(acc, stage, out_h, cid, sid, SR, SI)

    return k(dst2d)


def _sc_agg(src2d, dst2d, table, F):
    split = F == 32
    FW = 16 if split else F
    CH = _CH[F]
    GPW = EP // (CH * (NS if split else NW))
    tshape = (NC, NPAD, FW) if split else ((NPAD,) if F == 1 else (NPAD, F))
    mshape = (2, CH) if F == 1 else (2, CH, FW)
    oshape = (NC * NPAD,) if F == 1 else (NC * NPAD, FW)
    SR, SI = _STAGE[FW]
    sshape = (SR,) if SI == 1 else (2, SR, FW)
    ashape = (NPAD,) if F == 1 else (NPAD, FW)
    shared = [pltpu.VMEM_SHARED(ashape, jnp.float32)]
    if F == 1:
        shared.append(pltpu.VMEM_SHARED((NPAD,), jnp.float32))

    @functools.partial(
        pl.kernel,
        out_type=jax.ShapeDtypeStruct(oshape, jnp.float32),
        mesh=_MESH,
        compiler_params=pltpu.CompilerParams(use_tc_tiling_on_sc=False),
        scratch_types=shared + [
            pltpu.VMEM(sshape, jnp.float32),
            pltpu.VMEM((4, CH), jnp.int32),
            pltpu.VMEM((4, CH), jnp.int32),
            pltpu.VMEM(mshape, jnp.float32),
        ] + [pltpu.SemaphoreType.DMA] * 14,
    )
    def k(src_h, dst_h, tab_h, out_h, acc, *rest):
        if F == 1:
            tab_s, stage, src_v, dst_v, msg_v = rest[:5]
        else:
            stage, src_v, dst_v, msg_v = rest[:4]
        sems = rest[5:] if F == 1 else rest[4:]
        isem, gsem, ssem, rsem = sems[:4], sems[4:8], sems[8:12], sems[12:]
        wid, cid, sid = _wid_cid_sid()
        if F == 1:
            pltpu.sync_copy(tab_h.at[pl.ds(sid * RPS, RPS)], stage)
            pltpu.sync_copy(stage, tab_s.at[pl.ds(sid * RPS, RPS)])
            tab = tab_s
        elif split:
            tab = tab_h.at[cid]
        else:
            tab = tab_h
        _zero_acc(acc, stage, sid, FW, SR, SI)
        plsc.subcore_barrier()
        base = (sid if split else wid) * GPW

        def idx_load(q, g):
            pltpu.async_copy(src_h.at[base + g], src_v.at[q], isem[q])
            pltpu.async_copy(dst_h.at[base + g], dst_v.at[q], isem[q])

        def idx_wait(q):
            pltpu.make_async_copy(src_h.at[0], src_v.at[q], isem[q]).wait()
            pltpu.make_async_copy(dst_h.at[0], dst_v.at[q], isem[q]).wait()

        def ga_fire(q, m):
            pltpu.async_copy(tab.at[src_v.at[q]], msg_v.at[m], gsem[q])

        def ga_wait(q, m):
            pltpu.make_async_copy(tab.at[src_v.at[q]], msg_v.at[m],
                                  gsem[q]).wait()

        def sc_fire(q, m):
            pltpu.async_copy(msg_v.at[m], acc.at[dst_v.at[q]], ssem[q],
                             add=True)

        def sc_wait(q, m):
            pltpu.make_async_copy(msg_v.at[m], acc.at[dst_v.at[q]],
                                  ssem[q]).wait()

        def step(g, q, first, last):
            qn, m, mn = (q + 1) % 4, q % 2, (q + 1) % 2
            if not last:
                idx_load(qn, g + 1)
            ga_wait(q, m)
            sc_fire(q, m)
            if not first:
                sc_wait((q + 3) % 4, mn)
            if not last:
                idx_wait(qn)
                ga_fire(qn, mn)

        def prologue():
            idx_load(0, 0)
            idx_wait(0)
            ga_fire(0, 0)

        _run_pipeline(GPW, step, prologue)
        q = (GPW - 1) % 4
        sc_wait(q, q % 2)
        plsc.subcore_barrier()
        _read_acc(acc, stage, out_h, cid, sid, SR, SI, rsem)

    return k(src2d, dst2d, table)


def _d0_body(cntp, x, dinv_o, g1_o):
    c = cntp[...]
    deg = c[0] + c[1] + 1.0
    dinv = lax.rsqrt(deg)
    dinv_o[...] = dinv
    g1_o[...] = dinv * x[...]


def _d1_body(s1p, g1, dinv, W1c, b1c, g2_o):
    s = s1p[...]
    di = dinv[...]
    y = di * (s[0] + s[1] + g1[...])
    h = jnp.maximum(W1c[...] * y + b1c[...], 0.0)
    g2_o[...] = di * h


def _d2_body(s2p, g2, dinv, W2, b2c, g3_o):
    s = s2p[...]
    di = dinv[...]
    a = di * (s[0] + s[1] + g2[...])
    h = lax.dot_general(W2[...], a, (((0,), (0,)), ((), ())),
                        preferred_element_type=jnp.float32)
    h = jnp.maximum(h + b2c[...], 0.0)
    g3_o[...] = di * h


def _d3_body(s3p, g3, dinv, W3, b3c, batch, Wfc, bfc, out):
    di = dinv[...]
    a = di * (s3p[...] + g3[...])
    h = lax.dot_general(W3[...], a, (((0,), (0,)), ((), ())),
                        preferred_element_type=jnp.float32)
    h3 = jnp.maximum(h + b3c[...], 0.0)
    seg = lax.broadcasted_iota(jnp.int32, (64, 1), 0)
    B = (batch[...] == seg).astype(jnp.float32)
    sums = lax.dot_general(h3, B, (((1,), (1,)), ((), ())),
                           preferred_element_type=jnp.float32)
    ones = jnp.ones((1, NPAD), jnp.float32)
    cnts = lax.dot_general(ones, B, (((1,), (1,)), ((), ())),
                           preferred_element_type=jnp.float32)
    pooled = sums / jnp.maximum(cnts, 1.0)
    logits = lax.dot_general(pooled, Wfc[...], (((0,), (0,)), ((), ())),
                             preferred_element_type=jnp.float32) + bfc[...]
    m = jnp.max(logits, axis=1, keepdims=True)
    z = logits - m
    lse = jnp.log(jnp.sum(jnp.exp(z), axis=1, keepdims=True))
    out[...] = z - lse


def _tc(body, out_shape, *args):
    return pl.pallas_call(body, out_shape=out_shape)(*args)


def kernel(x, edge_index, batch, W1, b1, W2, b2, W3, b3, Wfc, bfc):
    ar = jnp.arange(EPAD, dtype=jnp.int32)
    srcp = jnp.concatenate([edge_index[0], ar % NPAD])
    dstp = jnp.concatenate([edge_index[1], N + (ar % PAD)])
    sv = {c: srcp.reshape(EP // c, c) for c in set(_CH.values())}
    dv = {c: dstp.reshape(EP // c, c) for c in set(_CH.values())}
    f32 = jnp.float32
    sds = jax.ShapeDtypeStruct

    cntp = _sc_counts(dv[_CH[1]])
    xp = jnp.pad(x[:, 0], (0, PAD)).reshape(1, NPAD)
    dinv, g1 = _tc(_d0_body,
                   (sds((1, NPAD), f32), sds((1, NPAD), f32)),
                   cntp.reshape(NC, 1, NPAD), xp)
    s1p = _sc_agg(sv[_CH[1]], dv[_CH[1]], g1.reshape(NPAD), 1)
    g2f = _tc(_d1_body, sds((16, NPAD), f32),
              s1p.reshape(NC, 1, NPAD), g1, dinv,
              W1.reshape(16, 1), b1.reshape(16, 1))
    s2p = _sc_agg(sv[_CH[16]], dv[_CH[16]], g2f.T, 16)
    g3f = _tc(_d2_body, sds((32, NPAD), f32),
              s2p.reshape(NC, NPAD, 16).transpose(0, 2, 1), g2f, dinv,
              W2, b2.reshape(32, 1))
    g3s = g3f.reshape(NC, 16, NPAD).transpose(0, 2, 1)
    s3p = _sc_agg(sv[_CH[32]], dv[_CH[32]], g3s, 32)
    s3pf = s3p.reshape(NC, NPAD, 16).transpose(0, 2, 1).reshape(32, NPAD)
    bp = jnp.pad(batch, (0, PAD), constant_values=64).reshape(1, NPAD)
    out = _tc(_d3_body, sds((64, 4), f32),
              s3pf, g3f, dinv,
              W3, b3.reshape(64, 1), bp, Wfc, bfc.reshape(1, 4))
    return out
